# baseline (speedup 1.0000x reference)
import functools

import jax
import jax.numpy as jnp
from jax.experimental import pallas as pl
from jax.experimental.pallas import tpu as pltpu

_F32_INF_BITS = 0x7F800000


def _search_iters(zs_ref, base, lo_ref, hi_ref, k, n_iter):
    n_l = zs_ref.shape[0] // 2
    d_lat = n_l * zs_ref.shape[2]

    def body(_, carry):
        lo, hi = carry
        mid = lo + ((hi - lo) >> 1)
        acc = jnp.zeros(zs_ref.shape[1:], jnp.int32)
        for lp in range(n_l):
            zb = jax.lax.bitcast_convert_type(zs_ref[base + lp], jnp.int32)
            acc = acc + jax.lax.shift_right_arithmetic(zb - mid, 31)
        cnt = d_lat + jnp.sum(acc, axis=1, keepdims=True)
        ge = cnt >= k
        eq = cnt == k
        lo = jnp.where(ge, mid, lo)
        hi = jnp.where(eq, mid + 1, jnp.where(ge, hi, mid))
        return lo, hi

    lo, hi = jax.lax.fori_loop(0, n_iter, body, (lo_ref[...], hi_ref[...]))
    lo_ref[...] = lo
    hi_ref[...] = hi


def _body(x_ref, w_enc_ref, b_enc_ref, z_in_ref, w_dec_ref, b_dec_ref,
          z_ref, zm_ref, o_ref, zs_ref, lo_ref, hi_ref, th_ref, acc_ref,
          *, n_t, n_l, k):
    t = pl.program_id(0)
    l = pl.program_id(1)
    bt = zs_ref.shape[1]

    @pl.when(t < n_t)
    def _encode():
        zt = jax.lax.dot_general(
            x_ref[...], w_enc_ref[...],
            dimension_numbers=(((1,), (1,)), ((), ())),
            preferred_element_type=jnp.float32,
        )
        zt = jnp.maximum(zt + b_enc_ref[0], 0.0)
        z_ref[...] = zt
        zs_ref[(t % 2) * n_l + l] = zt

    @pl.when((t >= 1) & (t <= n_t))
    def _search():
        @pl.when(l == 0)
        def _init():
            lo_ref[...] = jnp.zeros((bt, 1), jnp.int32)
            hi_ref[...] = jnp.full((bt, 1), _F32_INF_BITS, jnp.int32)

        avail = max(n_l - 1, 1)
        base, extra = 31 // avail, 31 % avail

        @pl.when(l < avail)
        def _iters():
            @pl.when(jnp.any(hi_ref[...] - lo_ref[...] > 1))
            def _go():
                n_iter = jnp.where(l == avail - 1, base + extra, base)
                _search_iters(zs_ref, ((t - 1) % 2) * n_l, lo_ref, hi_ref,
                              k, n_iter)

        @pl.when(l == avail - 1)
        def _emit():
            th_ref[(t - 1) % 2] = lo_ref[...]

    @pl.when(t >= 2)
    def _decode():
        z = z_in_ref[...]
        zb = jax.lax.bitcast_convert_type(z, jnp.int32)
        zm = jnp.where(zb >= th_ref[t % 2], z, 0.0)
        zm_ref[...] = zm

        @pl.when(l == 0)
        def _init_acc():
            acc_ref[...] = jnp.broadcast_to(b_dec_ref[...], acc_ref.shape)

        acc_ref[...] += jax.lax.dot_general(
            zm.astype(jnp.bfloat16), w_dec_ref[...],
            dimension_numbers=(((1,), (0,)), ((), ())),
            preferred_element_type=jnp.float32,
        )

        @pl.when(l == n_l - 1)
        def _emit_recon():
            o_ref[...] = acc_ref[...]


@functools.partial(jax.jit, static_argnames=("topk",))
def _run(x, W_enc, b_enc, W_dec, b_dec, topk=64):
    n_tok, d_in = x.shape
    d_lat = W_enc.shape[0]

    bt = 256 if n_tok % 256 == 0 else n_tok
    bl = 512 if d_lat % 512 == 0 else d_lat
    n_t, n_l = n_tok // bt, d_lat // bl
    b_enc3 = b_enc.reshape(n_l, 1, bl)
    b_dec2 = b_dec.reshape(1, d_in)
    x_bf = x.astype(jnp.bfloat16)
    w_enc_bf = W_enc.astype(jnp.bfloat16)
    w_dec_t = W_dec.T.astype(jnp.bfloat16)
    z_loop = jnp.zeros((n_tok + bt, d_lat), jnp.float32)

    tl = n_t - 1

    _, z_masked, recon = pl.pallas_call(
        functools.partial(_body, n_t=n_t, n_l=n_l, k=topk),
        grid=(n_t + 2, n_l),
        in_specs=[
            pl.BlockSpec((bt, d_in), lambda t, l: (jnp.minimum(t, tl), 0)),
            pl.BlockSpec((bl, d_in), lambda t, l: (l, 0)),
            pl.BlockSpec((1, 1, bl), lambda t, l: (l, 0, 0)),
            pl.BlockSpec((bt, bl), lambda t, l: (jnp.maximum(t - 2, 0), l)),
            pl.BlockSpec((bl, d_in), lambda t, l: (l, 0)),
            pl.BlockSpec((1, d_in), lambda t, l: (0, 0)),
        ],
        out_specs=[
            pl.BlockSpec((bt, bl), lambda t, l: (
                jnp.minimum(t, tl + 1), jnp.where(t > tl, n_l - 1, l))),
            pl.BlockSpec((bt, bl), lambda t, l: (
                jnp.maximum(t - 2, 0), jnp.where(t < 2, 0, l))),
            pl.BlockSpec((bt, d_in), lambda t, l: (jnp.maximum(t - 2, 0), 0)),
        ],
        out_shape=[
            jax.ShapeDtypeStruct((n_tok + bt, d_lat), jnp.float32),
            jax.ShapeDtypeStruct((n_tok, d_lat), jnp.float32),
            jax.ShapeDtypeStruct((n_tok, d_in), jnp.float32),
        ],
        scratch_shapes=[
            pltpu.VMEM((2 * n_l, bt, bl), jnp.float32),
            pltpu.VMEM((bt, 1), jnp.int32),
            pltpu.VMEM((bt, 1), jnp.int32),
            pltpu.VMEM((2, bt, 1), jnp.int32),
            pltpu.VMEM((bt, d_in), jnp.float32),
        ],
        input_output_aliases={3: 0},
    )(x_bf, w_enc_bf, b_enc3, z_loop, w_dec_t, b_dec2)

    return recon, z_masked


def kernel(x, W_enc, b_enc, W_dec, b_dec):
    return _run(x, W_enc, b_enc, W_dec, b_dec)


# speedup vs baseline: 1.1619x; 1.1619x over previous
import functools

import jax
import jax.numpy as jnp
from jax.experimental import pallas as pl
from jax.experimental.pallas import tpu as pltpu

_F32_INF_BITS = 0x7F800000


def _search_iters(zs_ref, base, lo_ref, hi_ref, k, n_iter):
    n_l = zs_ref.shape[0] // 2
    d_lat = n_l * zs_ref.shape[2]

    def body(_, carry):
        lo, hi = carry
        mid = lo + ((hi - lo) >> 1)
        acc = jnp.zeros(zs_ref.shape[1:], jnp.int32)
        for lp in range(n_l):
            zb = jax.lax.bitcast_convert_type(zs_ref[base + lp], jnp.int32)
            acc = acc + jax.lax.shift_right_arithmetic(zb - mid, 31)
        cnt = d_lat + jnp.sum(acc, axis=1, keepdims=True)
        ge = cnt >= k
        eq = cnt == k
        lo = jnp.where(ge, mid, lo)
        hi = jnp.where(eq, mid + 1, jnp.where(ge, hi, mid))
        return lo, hi

    lo, hi = jax.lax.fori_loop(0, n_iter, body, (lo_ref[...], hi_ref[...]))
    lo_ref[...] = lo
    hi_ref[...] = hi


def _search_iters16(zs16_ref, base, lo_ref, hi_ref, k, n_iter):
    n_l = zs16_ref.shape[0] // 2
    one = jnp.ones((), jnp.bfloat16)
    zero = jnp.zeros((), jnp.bfloat16)

    def body(_, carry):
        lo, hi = carry
        mid = lo + ((hi - lo) >> 1)
        mid16 = jax.lax.bitcast_convert_type(mid << 16, jnp.float32).astype(
            jnp.bfloat16)
        acc = jnp.zeros(zs16_ref.shape[1:], jnp.bfloat16)
        for lp in range(n_l):
            acc = acc + jnp.where(zs16_ref[base + lp] >= mid16, one, zero)
        cnt = jnp.sum(acc.astype(jnp.float32), axis=1,
                      keepdims=True).astype(jnp.int32)
        ge = cnt >= k
        eq = cnt == k
        lo = jnp.where(ge, mid, lo)
        hi = jnp.where(eq, mid, jnp.where(ge, hi, mid))
        return lo, hi

    lo, hi = jax.lax.fori_loop(0, n_iter, body, (lo_ref[...], hi_ref[...]))
    lo_ref[...] = lo
    hi_ref[...] = hi


def _phase_shift(lo_ref, hi_ref):
    lo = lo_ref[...] << 16
    hi = hi_ref[...] << 16
    lo_ref[...] = lo
    hi_ref[...] = jnp.where(hi == lo, lo + 1, hi)


_N16 = 15
_N32 = 16


def _encode_body(x_ref, w_ref, b_ref, z_ref, thr_ref, zs_ref, zs16_ref,
                 lo_ref, hi_ref, *, n_l, bl, k):
    t = pl.program_id(0)
    l = pl.program_id(1)
    zt = jax.lax.dot_general(
        x_ref[...], w_ref[...],
        dimension_numbers=(((1,), (1,)), ((), ())),
        preferred_element_type=jnp.float32,
    )
    zt = jnp.maximum(zt + b_ref[0], 0.0)
    z_ref[...] = zt
    zs_ref[(t % 2) * n_l + l] = zt
    tb = jax.lax.bitcast_convert_type(zt, jnp.int32) & jnp.int32(-65536)
    zs16_ref[(t % 2) * n_l + l] = jax.lax.bitcast_convert_type(
        tb, jnp.float32).astype(jnp.bfloat16)

    bt = zs_ref.shape[1]
    avail = max(n_l - 1, 1)
    if avail >= _N16 + _N32:
        last16 = _N16 - 1

        def sched(l):
            return (jnp.where(l < _N16, 1, 0),
                    jnp.where((l >= _N16) & (l < _N16 + _N32), 1, 0))
    elif avail == 1:
        last16 = 0

        def sched(l):
            return jnp.where(l == 0, _N16, 0), jnp.where(l == 0, _N32, 0)
    else:
        last16 = 0
        b32, e32 = _N32 // (avail - 1), _N32 % (avail - 1)

        def sched(l):
            return (jnp.where(l == 0, _N16, 0),
                    jnp.where((l > 0) & (l < avail),
                              b32 + jnp.where(l == avail - 1, e32, 0), 0))

    @pl.when(t > 0)
    def _search():
        @pl.when(l == 0)
        def _init():
            lo_ref[...] = jnp.zeros((bt, 1), jnp.int32)
            hi_ref[...] = jnp.full((bt, 1), _F32_INF_BITS >> 16, jnp.int32)

        n16, n32 = sched(l)
        par = ((t - 1) % 2) * n_l

        @pl.when((n16 > 0) & jnp.any(hi_ref[...] - lo_ref[...] > 1))
        def _go16():
            _search_iters16(zs16_ref, par, lo_ref, hi_ref, k, n16)

        @pl.when(l == last16)
        def _shift():
            _phase_shift(lo_ref, hi_ref)

        @pl.when((n32 > 0) & jnp.any(hi_ref[...] - lo_ref[...] > 1))
        def _go32():
            _search_iters(zs_ref, par, lo_ref, hi_ref, k, n32)

        @pl.when(l == avail - 1)
        def _emit():
            thr_ref[...] = lo_ref[...]

    @pl.when((t == pl.num_programs(0) - 1) & (l == n_l - 1))
    def _tail():
        lo_ref[...] = jnp.zeros((bt, 1), jnp.int32)
        hi_ref[...] = jnp.full((bt, 1), _F32_INF_BITS >> 16, jnp.int32)
        _search_iters16(zs16_ref, (t % 2) * n_l, lo_ref, hi_ref, k, _N16)
        _phase_shift(lo_ref, hi_ref)
        _search_iters(zs_ref, (t % 2) * n_l, lo_ref, hi_ref, k, _N32)
        thr_ref[...] = lo_ref[...]


def _decode_body(z_ref, thr_ref, w_ref, b_ref, o_ref, zm_ref, acc_ref, *, n_k):
    kk = pl.program_id(1)

    @pl.when(kk == 0)
    def _init():
        acc_ref[...] = jnp.broadcast_to(b_ref[...], acc_ref.shape)

    z = z_ref[...]
    zb = jax.lax.bitcast_convert_type(z, jnp.int32)
    zm = jnp.where(zb >= thr_ref[...], z, 0.0)
    zm_ref[...] = zm
    acc_ref[...] += jax.lax.dot_general(
        zm.astype(jnp.bfloat16), w_ref[...],
        dimension_numbers=(((1,), (0,)), ((), ())),
        preferred_element_type=jnp.float32,
    )

    @pl.when(kk == n_k - 1)
    def _emit():
        o_ref[...] = acc_ref[...]


@functools.partial(jax.jit, static_argnames=("topk",))
def _run(x, W_enc, b_enc, W_dec, b_dec, topk=64):
    n_tok, d_in = x.shape
    d_lat = W_enc.shape[0]

    bt = 256 if n_tok % 256 == 0 else n_tok
    bl = 512 if d_lat % 512 == 0 else d_lat
    n_t, n_l = n_tok // bt, d_lat // bl
    b_enc3 = b_enc.reshape(n_l, 1, bl)
    x_bf = x.astype(jnp.bfloat16)
    w_enc_bf = W_enc.astype(jnp.bfloat16)

    z_pre, thr = pl.pallas_call(
        functools.partial(_encode_body, n_l=n_l, bl=bl, k=topk),
        grid=(n_t, n_l),
        in_specs=[
            pl.BlockSpec((bt, d_in), lambda t, l: (t, 0)),
            pl.BlockSpec((bl, d_in), lambda t, l: (l, 0)),
            pl.BlockSpec((1, 1, bl), lambda t, l: (l, 0, 0)),
        ],
        out_specs=[
            pl.BlockSpec((bt, bl), lambda t, l: (t, l)),
            pl.BlockSpec(
                (bt, 1),
                lambda t, l: (
                    jnp.where((t == n_t - 1) & (l == n_l - 1),
                              t, jnp.maximum(t - 1, 0)), 0)),
        ],
        out_shape=[
            jax.ShapeDtypeStruct((n_tok, d_lat), jnp.float32),
            jax.ShapeDtypeStruct((n_tok, 1), jnp.int32),
        ],
        scratch_shapes=[
            pltpu.VMEM((2 * n_l, bt, bl), jnp.float32),
            pltpu.VMEM((2 * n_l, bt, bl), jnp.bfloat16),
            pltpu.VMEM((bt, 1), jnp.int32),
            pltpu.VMEM((bt, 1), jnp.int32),
        ],
    )(x_bf, w_enc_bf, b_enc3)

    w_dec_t = W_dec.T.astype(jnp.bfloat16)
    b_dec2 = b_dec.reshape(1, d_in)
    bt2 = 512 if n_tok % 512 == 0 else n_tok
    bk = 2048 if d_lat % 2048 == 0 else d_lat
    n_t2, n_k = n_tok // bt2, d_lat // bk

    recon, z_masked = pl.pallas_call(
        functools.partial(_decode_body, n_k=n_k),
        grid=(n_t2, n_k),
        in_specs=[
            pl.BlockSpec((bt2, bk), lambda t, kk: (t, kk)),
            pl.BlockSpec((bt2, 1), lambda t, kk: (t, 0)),
            pl.BlockSpec((bk, d_in), lambda t, kk: (kk, 0)),
            pl.BlockSpec((1, d_in), lambda t, kk: (0, 0)),
        ],
        out_specs=[
            pl.BlockSpec((bt2, d_in), lambda t, kk: (t, 0)),
            pl.BlockSpec((bt2, bk), lambda t, kk: (t, kk)),
        ],
        out_shape=[
            jax.ShapeDtypeStruct((n_tok, d_in), jnp.float32),
            jax.ShapeDtypeStruct((n_tok, d_lat), jnp.float32),
        ],
        scratch_shapes=[pltpu.VMEM((bt2, d_in), jnp.float32)],
    )(z_pre, thr, w_dec_t, b_dec2)

    return recon, z_masked


def kernel(x, W_enc, b_enc, W_dec, b_dec):
    return _run(x, W_enc, b_enc, W_dec, b_dec)


# speedup vs baseline: 1.1682x; 1.0054x over previous
import functools

import jax
import jax.numpy as jnp
from jax.experimental import pallas as pl
from jax.experimental.pallas import tpu as pltpu

_F32_INF_BITS = 0x7F800000


def _search_iters(zs_ref, base, lo_ref, hi_ref, k, n_iter):
    n_l = zs_ref.shape[0] // 2
    d_lat = n_l * zs_ref.shape[2]

    def body(_, carry):
        lo, hi = carry
        mid = lo + ((hi - lo) >> 1)
        acc = jnp.zeros(zs_ref.shape[1:], jnp.int32)
        for lp in range(n_l):
            zb = jax.lax.bitcast_convert_type(zs_ref[base + lp], jnp.int32)
            acc = acc + jax.lax.shift_right_arithmetic(zb - mid, 31)
        cnt = d_lat + jnp.sum(acc, axis=1, keepdims=True)
        ge = cnt >= k
        eq = cnt == k
        lo = jnp.where(ge, mid, lo)
        hi = jnp.where(eq, mid + 1, jnp.where(ge, hi, mid))
        return lo, hi

    lo, hi = jax.lax.fori_loop(0, n_iter, body, (lo_ref[...], hi_ref[...]))
    lo_ref[...] = lo
    hi_ref[...] = hi


def _search_iters16(zs16_ref, base, lo_ref, hi_ref, k, n_iter):
    n_l = zs16_ref.shape[0] // 2
    one = jnp.ones((), jnp.bfloat16)
    zero = jnp.zeros((), jnp.bfloat16)

    def body(_, carry):
        lo, hi = carry
        mid = lo + ((hi - lo) >> 1)
        mid16 = jax.lax.bitcast_convert_type(mid << 16, jnp.float32).astype(
            jnp.bfloat16)
        acc = jnp.zeros(zs16_ref.shape[1:], jnp.bfloat16)
        for lp in range(n_l):
            acc = acc + jnp.where(zs16_ref[base + lp] >= mid16, one, zero)
        cnt = jnp.sum(acc.astype(jnp.float32), axis=1,
                      keepdims=True).astype(jnp.int32)
        ge = cnt >= k
        eq = cnt == k
        lo = jnp.where(ge, mid, lo)
        hi = jnp.where(eq, mid, jnp.where(ge, hi, mid))
        return lo, hi

    lo, hi = jax.lax.fori_loop(0, n_iter, body, (lo_ref[...], hi_ref[...]))
    lo_ref[...] = lo
    hi_ref[...] = hi


def _phase_shift(lo_ref, hi_ref):
    lo = lo_ref[...] << 16
    hi = hi_ref[...] << 16
    lo_ref[...] = lo
    hi_ref[...] = jnp.where(hi == lo, lo + 1, hi)


_N16 = 15
_N32 = 16


def _encode_body(x_ref, w_ref, b_ref, zm_ref, zmb_ref, zs_ref, zs16_ref,
                 lo_ref, hi_ref, th_ref, *, n_t, n_l, bl, k):
    t = pl.program_id(0)
    l = pl.program_id(1)

    @pl.when(t >= 2)
    def _mask():
        z = zs_ref[(t % 2) * n_l + l]
        zb = jax.lax.bitcast_convert_type(z, jnp.int32)
        zm = jnp.where(zb >= th_ref[t % 2], z, 0.0)
        zm_ref[...] = zm
        zmb_ref[...] = zm.astype(jnp.bfloat16)

    @pl.when(t < n_t)
    def _encode():
        zt = jax.lax.dot_general(
            x_ref[...], w_ref[...],
            dimension_numbers=(((1,), (1,)), ((), ())),
            preferred_element_type=jnp.float32,
        )
        zt = jnp.maximum(zt + b_ref[0], 0.0)
        zs_ref[(t % 2) * n_l + l] = zt
        tb = jax.lax.bitcast_convert_type(zt, jnp.int32) & jnp.int32(-65536)
        zs16_ref[(t % 2) * n_l + l] = jax.lax.bitcast_convert_type(
            tb, jnp.float32).astype(jnp.bfloat16)

    bt = zs_ref.shape[1]
    avail = max(n_l - 1, 1)
    if avail >= _N16 + _N32:
        last16 = _N16 - 1

        def sched(l):
            return (jnp.where(l < _N16, 1, 0),
                    jnp.where((l >= _N16) & (l < _N16 + _N32), 1, 0))
    elif avail == 1:
        last16 = 0

        def sched(l):
            return jnp.where(l == 0, _N16, 0), jnp.where(l == 0, _N32, 0)
    else:
        last16 = 0
        b32, e32 = _N32 // (avail - 1), _N32 % (avail - 1)

        def sched(l):
            return (jnp.where(l == 0, _N16, 0),
                    jnp.where((l > 0) & (l < avail),
                              b32 + jnp.where(l == avail - 1, e32, 0), 0))

    @pl.when((t >= 1) & (t <= n_t))
    def _search():
        @pl.when(l == 0)
        def _init():
            lo_ref[...] = jnp.zeros((bt, 1), jnp.int32)
            hi_ref[...] = jnp.full((bt, 1), _F32_INF_BITS >> 16, jnp.int32)

        n16, n32 = sched(l)
        par = ((t - 1) % 2) * n_l

        @pl.when((n16 > 0) & jnp.any(hi_ref[...] - lo_ref[...] > 1))
        def _go16():
            _search_iters16(zs16_ref, par, lo_ref, hi_ref, k, n16)

        @pl.when(l == last16)
        def _shift():
            _phase_shift(lo_ref, hi_ref)

        @pl.when((n32 > 0) & jnp.any(hi_ref[...] - lo_ref[...] > 1))
        def _go32():
            _search_iters(zs_ref, par, lo_ref, hi_ref, k, n32)

        @pl.when(l == avail - 1)
        def _emit():
            th_ref[(t - 1) % 2] = lo_ref[...]


def _decode_body(z_ref, w_ref, b_ref, o_ref, acc_ref, *, n_k):
    kk = pl.program_id(1)

    @pl.when(kk == 0)
    def _init():
        acc_ref[...] = jnp.broadcast_to(b_ref[...], acc_ref.shape)

    acc_ref[...] += jax.lax.dot_general(
        z_ref[...], w_ref[...],
        dimension_numbers=(((1,), (0,)), ((), ())),
        preferred_element_type=jnp.float32,
    )

    @pl.when(kk == n_k - 1)
    def _emit():
        o_ref[...] = acc_ref[...]


@functools.partial(jax.jit, static_argnames=("topk",))
def _run(x, W_enc, b_enc, W_dec, b_dec, topk=64):
    n_tok, d_in = x.shape
    d_lat = W_enc.shape[0]

    bt = 256 if n_tok % 256 == 0 else n_tok
    bl = 512 if d_lat % 512 == 0 else d_lat
    n_t, n_l = n_tok // bt, d_lat // bl
    b_enc3 = b_enc.reshape(n_l, 1, bl)
    x_bf = x.astype(jnp.bfloat16)
    w_enc_bf = W_enc.astype(jnp.bfloat16)

    tl = n_t - 1
    z_masked, z_masked_bf = pl.pallas_call(
        functools.partial(_encode_body, n_t=n_t, n_l=n_l, bl=bl, k=topk),
        grid=(n_t + 2, n_l),
        in_specs=[
            pl.BlockSpec((bt, d_in), lambda t, l: (jnp.minimum(t, tl), 0)),
            pl.BlockSpec((bl, d_in), lambda t, l: (l, 0)),
            pl.BlockSpec((1, 1, bl), lambda t, l: (l, 0, 0)),
        ],
        out_specs=[
            pl.BlockSpec((bt, bl), lambda t, l: (
                jnp.maximum(t - 2, 0), jnp.where(t < 2, 0, l))),
            pl.BlockSpec((bt, bl), lambda t, l: (
                jnp.maximum(t - 2, 0), jnp.where(t < 2, 0, l))),
        ],
        out_shape=[
            jax.ShapeDtypeStruct((n_tok, d_lat), jnp.float32),
            jax.ShapeDtypeStruct((n_tok, d_lat), jnp.bfloat16),
        ],
        scratch_shapes=[
            pltpu.VMEM((2 * n_l, bt, bl), jnp.float32),
            pltpu.VMEM((2 * n_l, bt, bl), jnp.bfloat16),
            pltpu.VMEM((bt, 1), jnp.int32),
            pltpu.VMEM((bt, 1), jnp.int32),
            pltpu.VMEM((2, bt, 1), jnp.int32),
        ],
    )(x_bf, w_enc_bf, b_enc3)

    w_dec_t = W_dec.T.astype(jnp.bfloat16)
    b_dec2 = b_dec.reshape(1, d_in)
    bt2 = 512 if n_tok % 512 == 0 else n_tok
    bk = 2048 if d_lat % 2048 == 0 else d_lat
    n_t2, n_k = n_tok // bt2, d_lat // bk

    recon = pl.pallas_call(
        functools.partial(_decode_body, n_k=n_k),
        grid=(n_t2, n_k),
        in_specs=[
            pl.BlockSpec((bt2, bk), lambda t, kk: (t, kk)),
            pl.BlockSpec((bk, d_in), lambda t, kk: (kk, 0)),
            pl.BlockSpec((1, d_in), lambda t, kk: (0, 0)),
        ],
        out_specs=pl.BlockSpec((bt2, d_in), lambda t, kk: (t, 0)),
        out_shape=jax.ShapeDtypeStruct((n_tok, d_in), jnp.float32),
        scratch_shapes=[pltpu.VMEM((bt2, d_in), jnp.float32)],
    )(z_masked_bf, w_dec_t, b_dec2)

    return recon, z_masked


def kernel(x, W_enc, b_enc, W_dec, b_dec):
    return _run(x, W_enc, b_enc, W_dec, b_dec)


# speedup vs baseline: 1.1798x; 1.0100x over previous
import functools

import jax
import jax.numpy as jnp
from jax.experimental import pallas as pl
from jax.experimental.pallas import tpu as pltpu

_F32_INF_BITS = 0x7F800000


def _search_iters(zs_ref, base, lo_ref, hi_ref, k, n_iter):
    n_l = zs_ref.shape[0] // 2
    d_lat = n_l * zs_ref.shape[2]

    def body(_, carry):
        lo, hi = carry
        mid = lo + ((hi - lo) >> 1)
        acc = jnp.zeros(zs_ref.shape[1:], jnp.int32)
        for lp in range(n_l):
            zb = jax.lax.bitcast_convert_type(zs_ref[base + lp], jnp.int32)
            acc = acc + jax.lax.shift_right_arithmetic(zb - mid, 31)
        cnt = d_lat + jnp.sum(acc, axis=1, keepdims=True)
        ge = cnt >= k
        eq = cnt == k
        lo = jnp.where(ge, mid, lo)
        hi = jnp.where(eq, mid + 1, jnp.where(ge, hi, mid))
        return lo, hi

    lo, hi = jax.lax.fori_loop(0, n_iter, body, (lo_ref[...], hi_ref[...]))
    lo_ref[...] = lo
    hi_ref[...] = hi


def _search_iters16(zs16_ref, base, lo_ref, hi_ref, k, n_iter):
    n_l = zs16_ref.shape[0] // 2
    one = jnp.ones((), jnp.bfloat16)
    zero = jnp.zeros((), jnp.bfloat16)

    def body(_, carry):
        lo, hi = carry
        mid = lo + ((hi - lo) >> 1)
        mid16 = jax.lax.bitcast_convert_type(mid << 16, jnp.float32).astype(
            jnp.bfloat16)
        acc = jnp.zeros(zs16_ref.shape[1:], jnp.bfloat16)
        for lp in range(n_l):
            acc = acc + jnp.where(zs16_ref[base + lp] >= mid16, one, zero)
        cnt = jnp.sum(acc.astype(jnp.float32), axis=1,
                      keepdims=True).astype(jnp.int32)
        ge = cnt >= k
        eq = cnt == k
        lo = jnp.where(ge, mid, lo)
        hi = jnp.where(eq, mid, jnp.where(ge, hi, mid))
        return lo, hi

    lo, hi = jax.lax.fori_loop(0, n_iter, body, (lo_ref[...], hi_ref[...]))
    lo_ref[...] = lo
    hi_ref[...] = hi


def _phase_shift(lo_ref, hi_ref):
    lo = lo_ref[...] << 16
    hi = hi_ref[...] << 16
    lo_ref[...] = lo
    hi_ref[...] = jnp.where(hi == lo, lo + 1, hi)


_N16 = 15
_N32 = 16


def _encode_body(x_ref, w_ref, b_ref, zm_ref, zmb_ref, zs_ref, zs16_ref,
                 lo_ref, hi_ref, th_ref, *, n_t, n_l, bl, k):
    t = pl.program_id(0)
    l = pl.program_id(1)

    @pl.when(t >= 2)
    def _mask():
        z = zs_ref[(t % 2) * n_l + l]
        zb = jax.lax.bitcast_convert_type(z, jnp.int32)
        zm = jnp.where(zb >= th_ref[t % 2], z, 0.0)
        zm_ref[...] = zm
        zmb_ref[...] = zm.astype(jnp.bfloat16)

    @pl.when(t < n_t)
    def _encode():
        zt = jax.lax.dot_general(
            x_ref[...], w_ref[...],
            dimension_numbers=(((1,), (1,)), ((), ())),
            preferred_element_type=jnp.float32,
        )
        zt = jnp.maximum(zt + b_ref[0], 0.0)
        zs_ref[(t % 2) * n_l + l] = zt
        tb = jax.lax.bitcast_convert_type(zt, jnp.int32) & jnp.int32(-65536)
        zs16_ref[(t % 2) * n_l + l] = jax.lax.bitcast_convert_type(
            tb, jnp.float32).astype(jnp.bfloat16)

    bt = zs_ref.shape[1]
    avail = max(n_l - 1, 1)
    if avail >= _N16 + _N32:
        last16 = _N16 - 1

        def sched(l):
            return (jnp.where(l < _N16, 1, 0),
                    jnp.where((l >= _N16) & (l < _N16 + _N32), 1, 0))
    elif avail == 1:
        last16 = 0

        def sched(l):
            return jnp.where(l == 0, _N16, 0), jnp.where(l == 0, _N32, 0)
    else:
        last16 = 0
        b32, e32 = _N32 // (avail - 1), _N32 % (avail - 1)

        def sched(l):
            return (jnp.where(l == 0, _N16, 0),
                    jnp.where((l > 0) & (l < avail),
                              b32 + jnp.where(l == avail - 1, e32, 0), 0))

    @pl.when((t >= 1) & (t <= n_t))
    def _search():
        @pl.when(l == 0)
        def _init():
            lo_ref[...] = jnp.zeros((bt, 1), jnp.int32)
            hi_ref[...] = jnp.full((bt, 1), _F32_INF_BITS >> 16, jnp.int32)

        n16, n32 = sched(l)
        par = ((t - 1) % 2) * n_l

        @pl.when((n16 > 0) & jnp.any(hi_ref[...] - lo_ref[...] > 1))
        def _go16():
            _search_iters16(zs16_ref, par, lo_ref, hi_ref, k, n16)

        @pl.when(l == last16)
        def _shift():
            _phase_shift(lo_ref, hi_ref)

        @pl.when((n32 > 0) & jnp.any(hi_ref[...] - lo_ref[...] > 1))
        def _go32():
            _search_iters(zs_ref, par, lo_ref, hi_ref, k, n32)

        @pl.when(l == avail - 1)
        def _emit():
            th_ref[(t - 1) % 2] = jnp.broadcast_to(lo_ref[...], th_ref.shape[1:])


def _decode_body(z_ref, w_ref, b_ref, o_ref, acc_ref, *, n_k):
    kk = pl.program_id(1)

    @pl.when(kk == 0)
    def _init():
        acc_ref[...] = jnp.broadcast_to(b_ref[...], acc_ref.shape)

    acc_ref[...] += jax.lax.dot_general(
        z_ref[...], w_ref[...],
        dimension_numbers=(((1,), (0,)), ((), ())),
        preferred_element_type=jnp.float32,
    )

    @pl.when(kk == n_k - 1)
    def _emit():
        o_ref[...] = acc_ref[...]


@functools.partial(jax.jit, static_argnames=("topk",))
def _run(x, W_enc, b_enc, W_dec, b_dec, topk=64):
    n_tok, d_in = x.shape
    d_lat = W_enc.shape[0]

    bt = 256 if n_tok % 256 == 0 else n_tok
    bl = 512 if d_lat % 512 == 0 else d_lat
    n_t, n_l = n_tok // bt, d_lat // bl
    b_enc3 = b_enc.reshape(n_l, 1, bl)
    x_bf = x.astype(jnp.bfloat16)
    w_enc_bf = W_enc.astype(jnp.bfloat16)

    tl = n_t - 1
    z_masked, z_masked_bf = pl.pallas_call(
        functools.partial(_encode_body, n_t=n_t, n_l=n_l, bl=bl, k=topk),
        grid=(n_t + 2, n_l),
        in_specs=[
            pl.BlockSpec((bt, d_in), lambda t, l: (jnp.minimum(t, tl), 0)),
            pl.BlockSpec((bl, d_in), lambda t, l: (l, 0)),
            pl.BlockSpec((1, 1, bl), lambda t, l: (l, 0, 0)),
        ],
        out_specs=[
            pl.BlockSpec((bt, bl), lambda t, l: (
                jnp.maximum(t - 2, 0), jnp.where(t < 2, 0, l))),
            pl.BlockSpec((bt, bl), lambda t, l: (
                jnp.maximum(t - 2, 0), jnp.where(t < 2, 0, l))),
        ],
        out_shape=[
            jax.ShapeDtypeStruct((n_tok, d_lat), jnp.float32),
            jax.ShapeDtypeStruct((n_tok, d_lat), jnp.bfloat16),
        ],
        scratch_shapes=[
            pltpu.VMEM((2 * n_l, bt, bl), jnp.float32),
            pltpu.VMEM((2 * n_l, bt, bl), jnp.bfloat16),
            pltpu.VMEM((bt, 1), jnp.int32),
            pltpu.VMEM((bt, 1), jnp.int32),
            pltpu.VMEM((2, bt, bl), jnp.int32),
        ],
    )(x_bf, w_enc_bf, b_enc3)

    w_dec_t = W_dec.T.astype(jnp.bfloat16)
    b_dec2 = b_dec.reshape(1, d_in)
    bt2 = 512 if n_tok % 512 == 0 else n_tok
    bk = 2048 if d_lat % 2048 == 0 else d_lat
    n_t2, n_k = n_tok // bt2, d_lat // bk

    recon = pl.pallas_call(
        functools.partial(_decode_body, n_k=n_k),
        grid=(n_t2, n_k),
        in_specs=[
            pl.BlockSpec((bt2, bk), lambda t, kk: (t, kk)),
            pl.BlockSpec((bk, d_in), lambda t, kk: (kk, 0)),
            pl.BlockSpec((1, d_in), lambda t, kk: (0, 0)),
        ],
        out_specs=pl.BlockSpec((bt2, d_in), lambda t, kk: (t, 0)),
        out_shape=jax.ShapeDtypeStruct((n_tok, d_in), jnp.float32),
        scratch_shapes=[pltpu.VMEM((bt2, d_in), jnp.float32)],
    )(z_masked_bf, w_dec_t, b_dec2)

    return recon, z_masked


def kernel(x, W_enc, b_enc, W_dec, b_dec):
    return _run(x, W_enc, b_enc, W_dec, b_dec)
